# baseline (speedup 1.0000x reference)
import functools

import jax
import jax.numpy as jnp
from jax import lax
from jax.experimental import pallas as pl
from jax.experimental.pallas import tpu as pltpu
from jax.experimental.pallas import tpu_sc as plsc

N = 10000
D = 128
NPAD = 10240
NC = 2
NS = 16
NW = NC * NS
CH = 128
E = 320000
EPW = 10240
E_PAD = EPW * NW
NCHUNK = EPW // CH
ROWS_PER_TILE = NPAD // NS

EPW0 = 15360
EPW1 = 5120
NCH0 = EPW0 // CH
NCH1 = EPW1 // CH
E0 = EPW0 * NS
E_ALLOC = E0 + (NS - 1) * EPW1 + EPW0

_MESH = plsc.VectorSubcoreMesh(core_axis_name="c", subcore_axis_name="s")


@functools.partial(
    pl.kernel,
    out_type=jax.ShapeDtypeStruct((NC, 2, NPAD), jnp.float32),
    mesh=_MESH,
    scratch_types=[
        pltpu.VMEM_SHARED((NPAD,), jnp.float32),
        pltpu.VMEM_SHARED((NPAD,), jnp.float32),
        pltpu.VMEM((NCHUNK, CH), jnp.int32),
        pltpu.VMEM((NCHUNK, CH), jnp.int32),
        pltpu.VMEM((CH,), jnp.float32),
        pltpu.VMEM((ROWS_PER_TILE,), jnp.float32),
        pltpu.SemaphoreType.DMA,
        pltpu.SemaphoreType.DMA,
    ],
)
def _deg(src_hbm, dst_hbm, degp_hbm, dego_sh, degi_sh, isrc_v, idst_v,
         ones_v, zd_v, semo, semi):
    c = lax.axis_index("c")
    s = lax.axis_index("s")
    wid = c * NS + s

    def fill(i, _):
        zd_v[pl.ds(i * 16, 16)] = jnp.zeros((16,), jnp.float32)
        return 0

    lax.fori_loop(0, ROWS_PER_TILE // 16, fill, 0)
    for j in range(CH // 16):
        ones_v[pl.ds(j * 16, 16)] = jnp.ones((16,), jnp.float32)

    stripe = pl.ds(s * ROWS_PER_TILE, ROWS_PER_TILE)
    pltpu.sync_copy(src_hbm.at[wid], isrc_v)
    pltpu.sync_copy(dst_hbm.at[wid], idst_v)
    pltpu.sync_copy(zd_v, dego_sh.at[stripe])
    pltpu.sync_copy(zd_v, degi_sh.at[stripe])
    plsc.subcore_barrier()

    def body(g, _):
        pltpu.async_copy(ones_v, dego_sh.at[isrc_v.at[g]], semo, add=True)
        pltpu.async_copy(ones_v, degi_sh.at[idst_v.at[g]], semi, add=True)
        return 0

    lax.fori_loop(0, NCHUNK, body, 0)

    def drain(g, _):
        pltpu.make_async_copy(ones_v, dego_sh.at[pl.ds(0, CH)], semo).wait()
        pltpu.make_async_copy(ones_v, degi_sh.at[pl.ds(0, CH)], semi).wait()
        return 0

    lax.fori_loop(0, NCHUNK, drain, 0)
    plsc.subcore_barrier()

    pltpu.sync_copy(dego_sh.at[stripe], degp_hbm.at[c, 0, stripe])
    pltpu.sync_copy(degi_sh.at[stripe], degp_hbm.at[c, 1, stripe])


@functools.partial(
    pl.kernel,
    out_type=jax.ShapeDtypeStruct((NC, NPAD, D), jnp.float32),
    mesh=_MESH,
    scratch_types=[
        pltpu.VMEM_SHARED((NPAD, D), jnp.float32),
        pltpu.VMEM((NCHUNK, CH), jnp.int32),
        [pltpu.VMEM((CH,), jnp.int32) for _ in range(2)],
        [pltpu.VMEM((CH, D), jnp.float32) for _ in range(2)],
        [pltpu.SemaphoreType.DMA for _ in range(2)],
        [pltpu.SemaphoreType.DMA for _ in range(2)],
    ],
)
def _agg(h_hbm, src_hbm, dst_hbm, agg_hbm, agg_sh, idst_v, isrc, bufs,
         gsem, ssem):
    c = lax.axis_index("c")
    s = lax.axis_index("s")
    wid = c * NS + s

    pltpu.sync_copy(dst_hbm.at[wid], idst_v)

    def fill(i, _):
        for j in range(D // 16):
            bufs[0][i, pl.ds(j * 16, 16)] = jnp.zeros((16,), jnp.float32)
        return 0

    lax.fori_loop(0, CH, fill, 0)
    for k in range(ROWS_PER_TILE // CH):
        pltpu.sync_copy(
            bufs[0], agg_sh.at[pl.ds(s * ROWS_PER_TILE + k * CH, CH)]
        )

    def idxload(g, b):
        pltpu.sync_copy(src_hbm.at[wid, g], isrc[b])

    def gather(g, b):
        pltpu.async_copy(h_hbm.at[isrc[b]], bufs[b], gsem[b])

    def gwait(b):
        pltpu.make_async_copy(h_hbm.at[pl.ds(0, CH)], bufs[b], gsem[b]).wait()

    def swait(b):
        pltpu.make_async_copy(
            bufs[b], agg_sh.at[pl.ds(0, CH)], ssem[b]
        ).wait()

    for b in range(2):
        idxload(b, b)
        gather(b, b)
    plsc.subcore_barrier()

    def body(t, _):
        for b in range(2):
            g = 2 * t + b
            gwait(b)
            pltpu.async_copy(bufs[b], agg_sh.at[idst_v.at[g]], ssem[b],
                             add=True)
            swait(b)

            @pl.when(g + 2 < NCHUNK)
            def _():
                idxload(g + 2, b)
                gather(g + 2, b)

        return 0

    lax.fori_loop(0, NCHUNK // 2, body, 0)
    plsc.subcore_barrier()

    stripe = pl.ds(s * ROWS_PER_TILE, ROWS_PER_TILE)
    pltpu.sync_copy(agg_sh.at[stripe], agg_hbm.at[c, stripe])


def _h2_body(x_ref, w_ref, deg_ref, out_ref):
    norm = lax.rsqrt(jnp.maximum(deg_ref[...], 1.0))
    out_ref[...] = jnp.dot(
        x_ref[...], w_ref[...], preferred_element_type=jnp.float32
    ) * norm[:, None]


def _out_body(agg_ref, deg_ref, b_ref, out_ref):
    a = agg_ref[0] + agg_ref[1]
    norm = lax.rsqrt(jnp.maximum(deg_ref[...], 1.0))
    out_ref[...] = a * norm[:, None] + b_ref[...][None, :]


_BM = 2048


def kernel(x, edge_index, W, b):
    src = edge_index[0].astype(jnp.int32)
    dst = edge_index[1].astype(jnp.int32)
    e = src.shape[0]
    src_f = jnp.concatenate([src, jnp.full((E_ALLOC - e,), N, jnp.int32)])
    dst_f = jnp.concatenate(
        [dst, jnp.full((E_ALLOC - e,), NPAD - 1, jnp.int32)]
    )
    src_p = src_f[:E_PAD].reshape(NW, NCHUNK, CH)
    dst_p = dst_f[:E_PAD].reshape(NW, NCHUNK, CH)
    dst_c = dst_f.reshape(E_ALLOC // CH, CH)
    x_pad = jnp.pad(x, ((0, NPAD - x.shape[0]), (0, 0)))

    degp = _deg(src_p, dst_p)
    deg_out = degp[0, 0] + degp[1, 0]
    deg_in = degp[0, 1] + degp[1, 1]

    h2 = pl.pallas_call(
        _h2_body,
        grid=(NPAD // _BM, NC),
        in_specs=[
            pl.BlockSpec((_BM, D), lambda i, c: (i, 0)),
            pl.BlockSpec((D, D), lambda i, c: (0, 0)),
            pl.BlockSpec((_BM,), lambda i, c: (i,)),
        ],
        out_specs=pl.BlockSpec(
            (_BM, D), lambda i, c: (c * (NPAD // _BM) + i, 0)
        ),
        out_shape=jax.ShapeDtypeStruct((NC * NPAD, D), jnp.float32),
    )(x_pad, W, deg_out)

    src_a = src_p.at[NS:].add(NPAD)
    aggp = _agg(h2, src_a, dst_p)

    out = pl.pallas_call(
        _out_body,
        grid=(NPAD // _BM,),
        in_specs=[
            pl.BlockSpec((NC, _BM, D), lambda i: (0, i, 0)),
            pl.BlockSpec((_BM,), lambda i: (i,)),
            pl.BlockSpec((D,), lambda i: (0,)),
        ],
        out_specs=pl.BlockSpec((_BM, D), lambda i: (i, 0)),
        out_shape=jax.ShapeDtypeStruct((NPAD, D), jnp.float32),
    )(aggp, deg_in, b)

    return out[:x.shape[0]]


# speedup vs baseline: 1.1698x; 1.1698x over previous
import functools

import jax
import jax.numpy as jnp
from jax import lax
from jax.experimental import pallas as pl
from jax.experimental.pallas import tpu as pltpu
from jax.experimental.pallas import tpu_sc as plsc

N = 10000
D = 128
NPAD = 10240
NC = 2
NS = 16
NW = NC * NS
CH = 128
E = 320000
EPW = 10240
E_PAD = EPW * NW
NCHUNK = EPW // CH
ROWS_PER_TILE = NPAD // NS

EPW0 = 15360
EPW1 = 5120
NCH0 = EPW0 // CH
NCH1 = EPW1 // CH
E0 = EPW0 * NS
E_ALLOC = E0 + (NS - 1) * EPW1 + EPW0

_MESH = plsc.VectorSubcoreMesh(core_axis_name="c", subcore_axis_name="s")


@functools.partial(
    pl.kernel,
    out_type=jax.ShapeDtypeStruct((NC, 2, NPAD), jnp.float32),
    mesh=_MESH,
    scratch_types=[
        pltpu.VMEM_SHARED((NPAD,), jnp.float32),
        pltpu.VMEM_SHARED((NPAD,), jnp.float32),
        pltpu.VMEM((NCHUNK, CH), jnp.int32),
        pltpu.VMEM((NCHUNK, CH), jnp.int32),
        pltpu.VMEM((CH,), jnp.float32),
        pltpu.VMEM((ROWS_PER_TILE,), jnp.float32),
        pltpu.SemaphoreType.DMA,
        pltpu.SemaphoreType.DMA,
    ],
)
def _deg(src_hbm, dst_hbm, degp_hbm, dego_sh, degi_sh, isrc_v, idst_v,
         ones_v, zd_v, semo, semi):
    c = lax.axis_index("c")
    s = lax.axis_index("s")
    wid = c * NS + s

    def fill(i, _):
        zd_v[pl.ds(i * 16, 16)] = jnp.zeros((16,), jnp.float32)
        return 0

    lax.fori_loop(0, ROWS_PER_TILE // 16, fill, 0)
    for j in range(CH // 16):
        ones_v[pl.ds(j * 16, 16)] = jnp.ones((16,), jnp.float32)

    stripe = pl.ds(s * ROWS_PER_TILE, ROWS_PER_TILE)
    pltpu.sync_copy(src_hbm.at[wid], isrc_v)
    pltpu.sync_copy(dst_hbm.at[wid], idst_v)
    pltpu.sync_copy(zd_v, dego_sh.at[stripe])
    pltpu.sync_copy(zd_v, degi_sh.at[stripe])
    plsc.subcore_barrier()

    def body(g, _):
        pltpu.async_copy(ones_v, dego_sh.at[isrc_v.at[g]], semo, add=True)
        pltpu.async_copy(ones_v, degi_sh.at[idst_v.at[g]], semi, add=True)
        return 0

    lax.fori_loop(0, NCHUNK, body, 0)

    def drain(g, _):
        pltpu.make_async_copy(ones_v, dego_sh.at[pl.ds(0, CH)], semo).wait()
        pltpu.make_async_copy(ones_v, degi_sh.at[pl.ds(0, CH)], semi).wait()
        return 0

    lax.fori_loop(0, NCHUNK, drain, 0)
    plsc.subcore_barrier()

    pltpu.sync_copy(dego_sh.at[stripe], degp_hbm.at[c, 0, stripe])
    pltpu.sync_copy(degi_sh.at[stripe], degp_hbm.at[c, 1, stripe])


@functools.partial(
    pl.kernel,
    out_type=jax.ShapeDtypeStruct((NC, NPAD, D), jnp.float32),
    mesh=_MESH,
    scratch_types=[
        pltpu.VMEM_SHARED((NPAD, D), jnp.float32),
        pltpu.VMEM((NCHUNK, CH), jnp.int32),
        [pltpu.VMEM((CH,), jnp.int32) for _ in range(2)],
        [pltpu.VMEM((CH, D), jnp.float32) for _ in range(2)],
        [pltpu.SemaphoreType.DMA for _ in range(2)],
        [pltpu.SemaphoreType.DMA for _ in range(2)],
    ],
)
def _agg(h_hbm, src_hbm, dst_hbm, agg_hbm, agg_sh, idst_v, isrc, bufs,
         gsem, ssem):
    c = lax.axis_index("c")
    s = lax.axis_index("s")
    wid = c * NS + s

    pltpu.sync_copy(dst_hbm.at[wid], idst_v)

    def fill(i, _):
        for j in range(D // 16):
            bufs[0][i, pl.ds(j * 16, 16)] = jnp.zeros((16,), jnp.float32)
        return 0

    lax.fori_loop(0, CH, fill, 0)
    for k in range(ROWS_PER_TILE // CH):
        pltpu.sync_copy(
            bufs[0], agg_sh.at[pl.ds(s * ROWS_PER_TILE + k * CH, CH)]
        )

    def idxload(g, b):
        pltpu.sync_copy(src_hbm.at[wid, g], isrc[b])

    def gather(g, b):
        pltpu.async_copy(h_hbm.at[isrc[b]], bufs[b], gsem[b])

    def gwait(b):
        pltpu.make_async_copy(h_hbm.at[pl.ds(0, CH)], bufs[b], gsem[b]).wait()

    def swait(b):
        pltpu.make_async_copy(
            bufs[b], agg_sh.at[pl.ds(0, CH)], ssem[b]
        ).wait()

    for b in range(2):
        idxload(b, b)
        gather(b, b)
    plsc.subcore_barrier()

    def body(t, _):
        for b in range(2):
            g = 2 * t + b
            gwait(b)
            pltpu.async_copy(bufs[b], agg_sh.at[idst_v.at[g]], ssem[b],
                             add=True)
            swait(b)

            @pl.when(g + 2 < NCHUNK)
            def _():
                idxload(g + 2, b)
                gather(g + 2, b)

        return 0

    lax.fori_loop(0, NCHUNK // 2, body, 0)
    plsc.subcore_barrier()

    stripe = pl.ds(s * ROWS_PER_TILE, ROWS_PER_TILE)
    pltpu.sync_copy(agg_sh.at[stripe], agg_hbm.at[c, stripe])


def _h2_body(x_ref, w_ref, deg_ref, out_ref):
    norm = lax.rsqrt(jnp.maximum(deg_ref[...], 1.0))
    out_ref[...] = jnp.dot(
        x_ref[...], w_ref[...], preferred_element_type=jnp.float32
    ) * norm[:, None]


def _out_body(agg_ref, deg_ref, b_ref, out_ref):
    a = agg_ref[0] + agg_ref[1]
    norm = lax.rsqrt(jnp.maximum(deg_ref[...], 1.0))
    out_ref[...] = a * norm[:, None] + b_ref[...][None, :]


_BM = 2048


def kernel(x, edge_index, W, b):
    src = edge_index[0].astype(jnp.int32)
    dst = edge_index[1].astype(jnp.int32)
    e = src.shape[0]
    src_f = jnp.concatenate([src, jnp.full((E_ALLOC - e,), N, jnp.int32)])
    dst_f = jnp.concatenate(
        [dst, jnp.full((E_ALLOC - e,), NPAD - 1, jnp.int32)]
    )
    src_p = src_f[:E_PAD].reshape(NW, NCHUNK, CH)
    dst_p = dst_f[:E_PAD].reshape(NW, NCHUNK, CH)
    dst_c = dst_f.reshape(E_ALLOC // CH, CH)
    x_pad = jnp.pad(x, ((0, NPAD - x.shape[0]), (0, 0)))

    degp = _deg(src_p, dst_p)
    deg_out = degp[0, 0] + degp[1, 0]
    deg_in = degp[0, 1] + degp[1, 1]

    h2 = pl.pallas_call(
        _h2_body,
        grid=(NPAD // _BM,),
        in_specs=[
            pl.BlockSpec((_BM, D), lambda i: (i, 0)),
            pl.BlockSpec((D, D), lambda i: (0, 0)),
            pl.BlockSpec((_BM,), lambda i: (i,)),
        ],
        out_specs=pl.BlockSpec((_BM, D), lambda i: (i, 0)),
        out_shape=jax.ShapeDtypeStruct((NPAD, D), jnp.float32),
    )(x_pad, W, deg_out)

    aggp = _agg(h2, src_p, dst_p)

    out = pl.pallas_call(
        _out_body,
        grid=(NPAD // _BM,),
        in_specs=[
            pl.BlockSpec((NC, _BM, D), lambda i: (0, i, 0)),
            pl.BlockSpec((_BM,), lambda i: (i,)),
            pl.BlockSpec((D,), lambda i: (0,)),
        ],
        out_specs=pl.BlockSpec((_BM, D), lambda i: (i, 0)),
        out_shape=jax.ShapeDtypeStruct((NPAD, D), jnp.float32),
    )(aggp, deg_in, b)

    return out[:x.shape[0]]


# speedup vs baseline: 1.3068x; 1.1171x over previous
import functools

import jax
import jax.numpy as jnp
from jax import lax
from jax.experimental import pallas as pl
from jax.experimental.pallas import tpu as pltpu
from jax.experimental.pallas import tpu_sc as plsc

N = 10000
D = 128
NPAD = 10240
NC = 2
NS = 16
NW = NC * NS
CH = 128
E = 320000
EPW = 10240
E_PAD = EPW * NW
NCHUNK = EPW // CH
ROWS_PER_TILE = NPAD // NS

_MESH = plsc.VectorSubcoreMesh(core_axis_name="c", subcore_axis_name="s")


@functools.partial(
    pl.kernel,
    out_type=jax.ShapeDtypeStruct((NC, 2, NPAD), jnp.float32),
    mesh=_MESH,
    scratch_types=[
        pltpu.VMEM_SHARED((NPAD,), jnp.float32),
        pltpu.VMEM_SHARED((NPAD,), jnp.float32),
        pltpu.VMEM((NCHUNK, CH), jnp.int32),
        pltpu.VMEM((NCHUNK, CH), jnp.int32),
        pltpu.VMEM((CH,), jnp.float32),
        pltpu.VMEM((ROWS_PER_TILE,), jnp.float32),
        pltpu.SemaphoreType.DMA,
        pltpu.SemaphoreType.DMA,
    ],
)
def _deg(src_hbm, dst_hbm, degp_hbm, dego_sh, degi_sh, isrc_v, idst_v,
         ones_v, zd_v, semo, semi):
    c = lax.axis_index("c")
    s = lax.axis_index("s")
    wid = c * NS + s

    def fill(i, _):
        zd_v[pl.ds(i * 16, 16)] = jnp.zeros((16,), jnp.float32)
        return 0

    lax.fori_loop(0, ROWS_PER_TILE // 16, fill, 0)
    for j in range(CH // 16):
        ones_v[pl.ds(j * 16, 16)] = jnp.ones((16,), jnp.float32)

    stripe = pl.ds(s * ROWS_PER_TILE, ROWS_PER_TILE)
    pltpu.sync_copy(src_hbm.at[wid], isrc_v)
    pltpu.sync_copy(dst_hbm.at[wid], idst_v)
    pltpu.sync_copy(zd_v, dego_sh.at[stripe])
    pltpu.sync_copy(zd_v, degi_sh.at[stripe])
    plsc.subcore_barrier()

    def body(g, _):
        pltpu.async_copy(ones_v, dego_sh.at[isrc_v.at[g]], semo, add=True)
        pltpu.async_copy(ones_v, degi_sh.at[idst_v.at[g]], semi, add=True)
        return 0

    lax.fori_loop(0, NCHUNK, body, 0)

    def drain(g, _):
        pltpu.make_async_copy(ones_v, dego_sh.at[pl.ds(0, CH)], semo).wait()
        pltpu.make_async_copy(ones_v, degi_sh.at[pl.ds(0, CH)], semi).wait()
        return 0

    lax.fori_loop(0, NCHUNK, drain, 0)
    plsc.subcore_barrier()

    pltpu.sync_copy(dego_sh.at[stripe], degp_hbm.at[c, 0, stripe])
    pltpu.sync_copy(degi_sh.at[stripe], degp_hbm.at[c, 1, stripe])


@functools.partial(
    pl.kernel,
    out_type=jax.ShapeDtypeStruct((NC, NPAD, D), jnp.float32),
    mesh=_MESH,
    scratch_types=[
        pltpu.VMEM_SHARED((NPAD, D), jnp.float32),
        pltpu.VMEM((NCHUNK, CH), jnp.int32),
        [pltpu.VMEM((CH,), jnp.int32) for _ in range(2)],
        [pltpu.VMEM((CH, D), jnp.float32) for _ in range(2)],
        [pltpu.SemaphoreType.DMA for _ in range(2)],
        [pltpu.SemaphoreType.DMA for _ in range(2)],
    ],
)
def _agg(h_hbm, src_hbm, dst_hbm, agg_hbm, agg_sh, idst_v, isrc, bufs,
         gsem, ssem):
    c = lax.axis_index("c")
    s = lax.axis_index("s")
    wid = c * NS + s

    pltpu.sync_copy(dst_hbm.at[wid], idst_v)

    def fill(i, _):
        for j in range(D // 16):
            bufs[0][i, pl.ds(j * 16, 16)] = jnp.zeros((16,), jnp.float32)
        return 0

    lax.fori_loop(0, CH, fill, 0)
    for k in range(ROWS_PER_TILE // CH):
        pltpu.sync_copy(
            bufs[0], agg_sh.at[pl.ds(s * ROWS_PER_TILE + k * CH, CH)]
        )

    def idxload(g, b):
        pltpu.sync_copy(src_hbm.at[wid, g], isrc[b])

    def gather(g, b):
        pltpu.async_copy(h_hbm.at[isrc[b]], bufs[b], gsem[b])

    def gwait(b):
        pltpu.make_async_copy(h_hbm.at[pl.ds(0, CH)], bufs[b], gsem[b]).wait()

    def swait(b):
        pltpu.make_async_copy(
            bufs[b], agg_sh.at[pl.ds(0, CH)], ssem[b]
        ).wait()

    for b in range(2):
        idxload(b, b)
        gather(b, b)
    plsc.subcore_barrier()

    def body(t, _):
        for b in range(2):
            g = 2 * t + b
            gwait(b)
            pltpu.async_copy(bufs[b], agg_sh.at[idst_v.at[g]], ssem[b],
                             add=True)
            swait(b)

            @pl.when(g + 2 < NCHUNK)
            def _():
                idxload(g + 2, b)
                gather(g + 2, b)

        return 0

    lax.fori_loop(0, NCHUNK // 2, body, 0)
    plsc.subcore_barrier()

    stripe = pl.ds(s * ROWS_PER_TILE, ROWS_PER_TILE)
    pltpu.sync_copy(agg_sh.at[stripe], agg_hbm.at[c, stripe])


def _h2_body(x_ref, w_ref, deg_ref, out_ref):
    norm = lax.rsqrt(jnp.maximum(deg_ref[...], 1.0))
    out_ref[...] = jnp.dot(
        x_ref[...], w_ref[...], preferred_element_type=jnp.float32
    ) * norm[:, None]


def _out_body(agg_ref, deg_ref, b_ref, out_ref):
    a = agg_ref[0] + agg_ref[1]
    norm = lax.rsqrt(jnp.maximum(deg_ref[...], 1.0))
    out_ref[...] = a * norm[:, None] + b_ref[...][None, :]


_BM = 2048


def kernel(x, edge_index, W, b):
    src = edge_index[0].astype(jnp.int32)
    dst = edge_index[1].astype(jnp.int32)
    e = src.shape[0]
    src_f = jnp.concatenate([src, jnp.full((E_PAD - e,), N, jnp.int32)])
    dst_f = jnp.concatenate([dst, jnp.full((E_PAD - e,), NPAD - 1, jnp.int32)])
    src_p = src_f.reshape(NW, NCHUNK, CH)
    dst_p = dst_f.reshape(NW, NCHUNK, CH)
    x_pad = jnp.pad(x, ((0, NPAD - x.shape[0]), (0, 0)))

    degp = _deg(src_p, dst_p)
    deg_out = degp[0, 0] + degp[1, 0]
    deg_in = degp[0, 1] + degp[1, 1]

    h2 = pl.pallas_call(
        _h2_body,
        grid=(NPAD // _BM,),
        in_specs=[
            pl.BlockSpec((_BM, D), lambda i: (i, 0)),
            pl.BlockSpec((D, D), lambda i: (0, 0)),
            pl.BlockSpec((_BM,), lambda i: (i,)),
        ],
        out_specs=pl.BlockSpec((_BM, D), lambda i: (i, 0)),
        out_shape=jax.ShapeDtypeStruct((NPAD, D), jnp.float32),
    )(x_pad, W, deg_out)

    aggp = _agg(h2, src_p, dst_p)

    out = pl.pallas_call(
        _out_body,
        grid=(NPAD // _BM,),
        in_specs=[
            pl.BlockSpec((NC, _BM, D), lambda i: (0, i, 0)),
            pl.BlockSpec((_BM,), lambda i: (i,)),
            pl.BlockSpec((D,), lambda i: (0,)),
        ],
        out_specs=pl.BlockSpec((_BM, D), lambda i: (i, 0)),
        out_shape=jax.ShapeDtypeStruct((NPAD, D), jnp.float32),
    )(aggp, deg_in, b)

    return out[:x.shape[0]]
